# baseline (speedup 1.0000x reference)
import functools

import jax
import jax.numpy as jnp
from jax import lax
from jax.experimental import pallas as pl
from jax.experimental.pallas import tpu as pltpu
from jax.experimental.pallas import tpu_sc as plsc

BATCH = 16384
D = 16
OUT_W = 3 * D
TAG_V = 100000

_info = plsc.get_sparse_core_info()
NC, NS = _info.num_cores, _info.num_subcores
NW = NC * NS
BPW = BATCH // NW

DEPTH = 8
GRP = 16
NGRP = BPW // GRP

_mesh = plsc.VectorSubcoreMesh(core_axis_name="c", subcore_axis_name="s")


@functools.partial(
    pl.kernel,
    mesh=_mesh,
    out_type=jax.ShapeDtypeStruct((BATCH * OUT_W,), jnp.float32),
    compiler_params=pltpu.CompilerParams(needs_layout_passes=False),
    scratch_types=[
        pltpu.VMEM((BPW,), jnp.int32),
        pltpu.VMEM((BPW,), jnp.int32),
        pltpu.VMEM((BPW,), jnp.int32),
        pltpu.VMEM((DEPTH, D, 128), jnp.float32),
        pltpu.VMEM((DEPTH, D, 128), jnp.float32),
        pltpu.VMEM((DEPTH, D, 128), jnp.float32),
        pltpu.VMEM((BPW * OUT_W,), jnp.float32),
        [pltpu.SemaphoreType.DMA] * DEPTH,
        [pltpu.SemaphoreType.DMA] * DEPTH,
        [pltpu.SemaphoreType.DMA] * DEPTH,
        pltpu.SemaphoreType.DMA,
    ],
)
def _gather3(
    uid, iid, tid, ut, it, tt, out,
    uix, iix, tix, u_tiles, i_tiles, t_tiles, cat_v,
    usems, isems, tsems, fsem,
):
    sid = lax.axis_index("s")
    wid = sid * NC + lax.axis_index("c")
    base = wid * BPW

    for ids, ivec in ((uid, uix), (iid, iix), (tid, tix)):
        pltpu.sync_copy(ids.at[pl.ds(base, BPW)], ivec)
    d_iota = lax.iota(jnp.int32, 16)

    tables = (
        (ut, uix, u_tiles, usems, 0),
        (it, iix, i_tiles, isems, 1),
        (tt, tix, t_tiles, tsems, 2),
    )

    def fire(tab, tiles, sems, col, slot):
        pltpu.async_copy(
            tab.at[:, pl.ds(pl.multiple_of(col, 128), 128)],
            tiles.at[slot],
            sems[slot],
        )

    for tab, ivec, tiles, sems, _t in tables:
        v0 = ivec[pl.ds(0, GRP)]
        cols = (v0 >> 7) * 128
        for s in range(DEPTH):
            fire(tab, tiles, sems, cols[s], s)

    def ring_body(g):
        vgs = []
        for tab, ivec, tiles, sems, _t in tables:
            vg = ivec[pl.ds(g * GRP, GRP)]
            vgs.append((vg & 127, (vg >> 7) * 128))
        vns = []
        for tab, ivec, tiles, sems, _t in tables:
            vn = ivec[pl.ds(jnp.minimum(g + 1, NGRP - 1) * GRP, GRP)]
            vns.append((vn >> 7) * 128)

        for s in range(GRP):
            j = g * GRP + s
            for (tab, ivec, tiles, sems, t), (lanes, cols), ncols in zip(
                tables, vgs, vns
            ):
                slot = s % DEPTH
                pltpu.make_async_copy(
                    tab.at[:, pl.ds(0, 128)], tiles.at[slot], sems[slot]
                ).wait()
                lane = jnp.broadcast_to(lanes[s], (16,))
                row = plsc.load_gather(tiles.at[slot], [d_iota, lane])
                cat_v[pl.ds(j * OUT_W + t * D, D)] = row
                col = cols[s + DEPTH] if s + DEPTH < GRP else ncols[s + DEPTH - GRP]

                @pl.when(j + DEPTH < BPW)
                def _():
                    fire(tab, tiles, sems, col, slot)

    pl.loop(0, NGRP)(ring_body)

    pltpu.sync_copy(cat_v, out.at[pl.ds(base * OUT_W, BPW * OUT_W)])


def kernel(user_id, item_id, tag_id, user_table, item_table, tag_table):
    flat = _gather3(
        user_id, item_id, tag_id,
        user_table.T, item_table.T, tag_table.T,
    )
    return flat.reshape(BATCH, OUT_W)


# speedup vs baseline: 1.1085x; 1.1085x over previous
import functools

import jax
import jax.numpy as jnp
from jax import lax
from jax.experimental import pallas as pl
from jax.experimental.pallas import tpu as pltpu
from jax.experimental.pallas import tpu_sc as plsc

BATCH = 16384
D = 16
OUT_W = 3 * D

_info = plsc.get_sparse_core_info()
NC, NS = _info.num_cores, _info.num_subcores
NW = NC * NS
BPW = BATCH // NW

NBUF = 16
NGRP = BPW // NBUF

_mesh = plsc.VectorSubcoreMesh(core_axis_name="c", subcore_axis_name="s")


@functools.partial(
    pl.kernel,
    mesh=_mesh,
    out_type=jax.ShapeDtypeStruct((OUT_W, BATCH), jnp.float32),
    compiler_params=pltpu.CompilerParams(needs_layout_passes=False),
    scratch_types=[
        pltpu.VMEM((BPW,), jnp.int32),
        pltpu.VMEM((BPW,), jnp.int32),
        pltpu.VMEM((BPW,), jnp.int32),
        pltpu.VMEM((NBUF, D, 128), jnp.float32),
        pltpu.VMEM((OUT_W, BPW), jnp.float32),
        [pltpu.SemaphoreType.DMA] * NBUF,
    ],
)
def _gather3(uid, iid, tid, ut, it, tt, out, uix, iix, tix, tile_v, cat_v, sems):
    wid = lax.axis_index("s") * NC + lax.axis_index("c")
    base = wid * BPW
    for ids, ivec in ((uid, uix), (iid, iix), (tid, tix)):
        pltpu.sync_copy(ids.at[pl.ds(base, BPW)], ivec)
    d_iota = lax.iota(jnp.int32, 16)

    for t, (tab, ivec) in enumerate(((ut, uix), (it, iix), (tt, tix))):

        def fire_group(g, tab=tab, ivec=ivec):
            vg = ivec[pl.ds(g * NBUF, NBUF)]
            cols = (vg >> 7) * 128
            for s in range(NBUF):
                col = pl.multiple_of(cols[s], 128)
                pltpu.async_copy(
                    tab.at[:, pl.ds(col, 128)], tile_v.at[s], sems[s]
                )

        fire_group(0)

        def ring_body(g, tab=tab, t=t, ivec=ivec):
            vg = ivec[pl.ds(g * NBUF, NBUF)]
            lanes = vg & 127

            def extract(s):
                j = g * NBUF + s
                lane = jnp.broadcast_to(lanes[s], (16,))
                row = plsc.load_gather(tile_v.at[s], [d_iota, lane])
                jv = jnp.broadcast_to(jnp.int32(0) + j, (16,))
                plsc.store_scatter(cat_v, [t * D + d_iota, jv], row)

            def wait_slot(s):
                pltpu.make_async_copy(
                    tab.at[:, pl.ds(0, 128)], tile_v.at[s], sems[s]
                ).wait()

            @pl.when(g + 1 < NGRP)
            def _():
                vn = ivec[pl.ds((g + 1) * NBUF, NBUF)]
                cols = (vn >> 7) * 128
                for s in range(NBUF):
                    wait_slot(s)
                    extract(s)
                    col = pl.multiple_of(cols[s], 128)
                    pltpu.async_copy(
                        tab.at[:, pl.ds(col, 128)], tile_v.at[s], sems[s]
                    )

            @pl.when(g + 1 >= NGRP)
            def _():
                for s in range(NBUF):
                    wait_slot(s)
                    extract(s)

        pl.loop(0, NGRP)(ring_body)

    pltpu.sync_copy(
        cat_v, out.at[pl.ds(0, OUT_W), pl.ds(base, BPW)]
    )


def kernel(user_id, item_id, tag_id, user_table, item_table, tag_table):
    out_t = _gather3(
        user_id, item_id, tag_id,
        user_table.T, item_table.T, tag_table.T,
    )
    return out_t.T
